# baseline (speedup 1.0000x reference)
import jax
import jax.numpy as jnp
from jax.experimental import pallas as pl
from jax.experimental.pallas import tpu as pltpu


def _concat_body(n_u, n_i, u_ref, i_ref, o_ref, sem_u, sem_i):
    u_copy = pltpu.make_async_copy(u_ref, o_ref.at[pl.ds(0, n_u), :], sem_u)
    i_copy = pltpu.make_async_copy(i_ref, o_ref.at[pl.ds(n_u, n_i), :], sem_i)
    u_copy.start()
    i_copy.start()
    u_copy.wait()
    i_copy.wait()


def kernel(user_emb, item_emb):
    n_u, emb = user_emb.shape
    n_i, _ = item_emb.shape
    import functools

    return pl.pallas_call(
        functools.partial(_concat_body, n_u, n_i),
        out_shape=jax.ShapeDtypeStruct((n_u + n_i, emb), user_emb.dtype),
        in_specs=[
            pl.BlockSpec(memory_space=pl.ANY),
            pl.BlockSpec(memory_space=pl.ANY),
        ],
        out_specs=pl.BlockSpec(memory_space=pl.ANY),
        scratch_shapes=[pltpu.SemaphoreType.DMA, pltpu.SemaphoreType.DMA],
    )(user_emb, item_emb)


# speedup vs baseline: 16.1280x; 16.1280x over previous
import functools

import jax
import jax.numpy as jnp
from jax.experimental import pallas as pl
from jax.experimental.pallas import tpu as pltpu

_CHUNK_ROWS = 10000
_NBUF = 8
_LOOKAHEAD = 4


def _concat_body(n_u, n_i, ch, u_ref, i_ref, o_ref, buf, in_sems, out_sems):
    nu = n_u // ch
    total = nu + n_i // ch

    def src_dst(idx):
        if idx < nu:
            return (
                u_ref.at[pl.ds(idx * ch, ch), :],
                o_ref.at[pl.ds(idx * ch, ch), :],
            )
        c = idx - nu
        return (
            i_ref.at[pl.ds(c * ch, ch), :],
            o_ref.at[pl.ds(n_u + c * ch, ch), :],
        )

    def in_copy(idx):
        src, _ = src_dst(idx)
        return pltpu.make_async_copy(src, buf.at[idx % _NBUF], in_sems.at[idx % _NBUF])

    def out_copy(idx):
        _, dst = src_dst(idx)
        return pltpu.make_async_copy(buf.at[idx % _NBUF], dst, out_sems.at[idx % _NBUF])

    for idx in range(total):
        if idx >= _NBUF:
            out_copy(idx - _NBUF).wait()
        in_copy(idx).start()
        if idx >= _LOOKAHEAD:
            j = idx - _LOOKAHEAD
            in_copy(j).wait()
            out_copy(j).start()
    for idx in range(total - _LOOKAHEAD, total):
        in_copy(idx).wait()
        out_copy(idx).start()
    for idx in range(max(0, total - _NBUF), total):
        out_copy(idx).wait()


def kernel(user_emb, item_emb):
    n_u, emb = user_emb.shape
    n_i, _ = item_emb.shape

    return pl.pallas_call(
        functools.partial(_concat_body, n_u, n_i, _CHUNK_ROWS),
        out_shape=jax.ShapeDtypeStruct((n_u + n_i, emb), user_emb.dtype),
        in_specs=[
            pl.BlockSpec(memory_space=pl.ANY),
            pl.BlockSpec(memory_space=pl.ANY),
        ],
        out_specs=pl.BlockSpec(memory_space=pl.ANY),
        scratch_shapes=[
            pltpu.VMEM((_NBUF, _CHUNK_ROWS, emb), jnp.float32),
            pltpu.SemaphoreType.DMA((_NBUF,)),
            pltpu.SemaphoreType.DMA((_NBUF,)),
        ],
    )(user_emb, item_emb)
